# Initial kernel scaffold; baseline (speedup 1.0000x reference)
import jax
import jax.numpy as jnp
from jax.experimental import pallas as pl


def kernel(x, emb, emb2):
    raise NotImplementedError("write your pallas kernel here")


# speedup vs baseline: 355.7654x; 355.7654x over previous
import functools

import jax
import jax.numpy as jnp
from jax import lax
from jax.experimental import pallas as pl
from jax.experimental.pallas import tpu as pltpu
from jax.experimental.pallas import tpu_sc as plsc

L = 16
NC = 2
NS = 16
NW = NC * NS

B, SEQ = 16384, 200
TOTAL = B * SEQ
PER_W = TOTAL // NW
CHUNK = 12800
NCHUNK = PER_W // CHUNK
VECS = CHUNK // L


@functools.partial(
    pl.kernel,
    out_type=jax.ShapeDtypeStruct((NW, L), jnp.float32),
    mesh=plsc.VectorSubcoreMesh(core_axis_name="c", subcore_axis_name="s"),
    compiler_params=pltpu.CompilerParams(needs_layout_passes=False),
    scratch_types=[
        pltpu.VMEM((CHUNK,), jnp.int32),
        pltpu.VMEM((CHUNK,), jnp.int32),
        pltpu.VMEM((L, L), jnp.float32),
        pltpu.VMEM((L, L), jnp.float32),
        pltpu.VMEM((L,), jnp.float32),
        pltpu.VMEM((1, L), jnp.float32),
        pltpu.SemaphoreType.DMA,
        pltpu.SemaphoreType.DMA,
    ],
)
def _sc_sum(x_hbm, ea_hbm, eb_hbm, out_hbm,
            buf0, buf1, tab_a, tab_b, s_ref, acc_ref, sem0, sem1):
    cid = lax.axis_index("c")
    sid = lax.axis_index("s")
    wid = sid * NC + cid
    base = wid * PER_W

    pltpu.sync_copy(ea_hbm, tab_a)
    pltpu.sync_copy(eb_hbm, tab_b)
    s = jnp.zeros((L,), jnp.float32)
    for k in range(L):
        s = s + tab_a[k] + tab_b[k]
    s_ref[...] = s

    bufs = (buf0, buf1)
    sems = (sem0, sem1)

    def dma(c, buf, sem):
        return pltpu.make_async_copy(
            x_hbm.at[pl.ds(base + c * CHUNK, CHUNK)], buf, sem)

    dma(0, buf0, sem0).start()
    acc = jnp.zeros((L,), jnp.float32)
    for c in range(NCHUNK):
        buf, sem = bufs[c % 2], sems[c % 2]
        if c + 1 < NCHUNK:
            dma(c + 1, bufs[(c + 1) % 2], sems[(c + 1) % 2]).start()
        dma(c, buf, sem).wait()

        def body(i, a, buf=buf):
            idx = buf[pl.ds(i * L, L)]
            return a + plsc.load_gather(s_ref, [idx])

        acc = lax.fori_loop(0, VECS, body, acc, unroll=8)

    acc_ref[0, :] = acc
    pltpu.sync_copy(acc_ref, out_hbm.at[pl.ds(wid, 1)])


def kernel(x, emb, emb2):
    xf = x.reshape(TOTAL).astype(jnp.int32)
    ea = jnp.zeros((L, L), jnp.float32).at[:10, :10].set(emb.T)
    eb = jnp.zeros((L, L), jnp.float32).at[:10, :10].set(emb2.T)
    partials = _sc_sum(xf, ea, eb)
    return jnp.sum(partials)


# speedup vs baseline: 506.3700x; 1.4233x over previous
import functools

import jax
import jax.numpy as jnp
from jax import lax
from jax.experimental import pallas as pl
from jax.experimental.pallas import tpu as pltpu
from jax.experimental.pallas import tpu_sc as plsc

L = 16
NC = 2
NS = 16
NW = NC * NS

B, SEQ = 16384, 200
ROWS_W = B // NW
RCHUNK = 64
NCHUNK = ROWS_W // RCHUNK
NFULL = SEQ // L
TAIL = SEQ - NFULL * L


@functools.partial(
    pl.kernel,
    out_type=jax.ShapeDtypeStruct((NW, L), jnp.float32),
    mesh=plsc.VectorSubcoreMesh(core_axis_name="c", subcore_axis_name="s"),
    compiler_params=pltpu.CompilerParams(needs_layout_passes=False),
    scratch_types=[
        pltpu.VMEM((RCHUNK, SEQ), jnp.int32),
        pltpu.VMEM((RCHUNK, SEQ), jnp.int32),
        pltpu.VMEM((L, L), jnp.float32),
        pltpu.VMEM((L, L), jnp.float32),
        pltpu.VMEM((L,), jnp.float32),
        pltpu.VMEM((1, L), jnp.float32),
        pltpu.SemaphoreType.DMA,
        pltpu.SemaphoreType.DMA,
    ],
)
def _sc_sum(x_hbm, ea_hbm, eb_hbm, out_hbm,
            buf0, buf1, tab_a, tab_b, s_ref, acc_ref, sem0, sem1):
    cid = lax.axis_index("c")
    sid = lax.axis_index("s")
    wid = sid * NC + cid
    row0 = wid * ROWS_W

    pltpu.sync_copy(ea_hbm, tab_a)
    pltpu.sync_copy(eb_hbm, tab_b)
    s = jnp.zeros((L,), jnp.float32)
    for k in range(L):
        s = s + tab_a[k] + tab_b[k]
    s_ref[...] = s

    tail_keep = lax.iota(jnp.int32, L) >= (L - TAIL)
    zeros = jnp.zeros((L,), jnp.float32)

    bufs = (buf0, buf1)
    sems = (sem0, sem1)

    def dma(c, buf, sem):
        return pltpu.make_async_copy(
            x_hbm.at[pl.ds(row0 + c * RCHUNK, RCHUNK), :], buf, sem)

    dma(0, buf0, sem0).start()
    acc = jnp.zeros((L,), jnp.float32)
    for c in range(NCHUNK):
        buf, sem = bufs[c % 2], sems[c % 2]
        if c + 1 < NCHUNK:
            dma(c + 1, bufs[(c + 1) % 2], sems[(c + 1) % 2]).start()
        dma(c, buf, sem).wait()

        def body(r, a, buf=buf):
            for j in range(NFULL):
                idx = buf[r, pl.ds(j * L, L)]
                a = a + plsc.load_gather(s_ref, [idx])
            idx = buf[r, pl.ds(SEQ - L, L)]
            vals = plsc.load_gather(s_ref, [idx])
            return a + jnp.where(tail_keep, vals, zeros)

        acc = lax.fori_loop(0, RCHUNK, body, acc)

    acc_ref[0, :] = acc
    pltpu.sync_copy(acc_ref, out_hbm.at[pl.ds(wid, 1)])


def kernel(x, emb, emb2):
    xi = x.astype(jnp.int32)
    ea = jnp.zeros((L, L), jnp.float32).at[:10, :10].set(emb.T)
    eb = jnp.zeros((L, L), jnp.float32).at[:10, :10].set(emb2.T)
    partials = _sc_sum(xi, ea, eb)
    return jnp.sum(partials)


# speedup vs baseline: 713.8866x; 1.4098x over previous
import functools

import jax
import jax.numpy as jnp
from jax import lax
from jax.experimental import pallas as pl
from jax.experimental.pallas import tpu as pltpu
from jax.experimental.pallas import tpu_sc as plsc

L = 16
NC = 2
NS = 16
NW = NC * NS

B, SEQ = 16384, 200
COLS_W = B // NW
RCHUNK = 40
NCHUNK = SEQ // RCHUNK
VROW = COLS_W // L


@functools.partial(
    pl.kernel,
    out_type=jax.ShapeDtypeStruct((NW, L), jnp.float32),
    mesh=plsc.VectorSubcoreMesh(core_axis_name="c", subcore_axis_name="s"),
    compiler_params=pltpu.CompilerParams(needs_layout_passes=False),
    scratch_types=[
        pltpu.VMEM((RCHUNK, COLS_W), jnp.int32),
        pltpu.VMEM((RCHUNK, COLS_W), jnp.int32),
        pltpu.VMEM((L, L), jnp.float32),
        pltpu.VMEM((L, L), jnp.float32),
        pltpu.VMEM((L,), jnp.float32),
        pltpu.VMEM((1, L), jnp.float32),
        pltpu.SemaphoreType.DMA,
        pltpu.SemaphoreType.DMA,
    ],
)
def _sc_sum(xt_hbm, ea_hbm, eb_hbm, out_hbm,
            buf0, buf1, tab_a, tab_b, s_ref, acc_ref, sem0, sem1):
    cid = lax.axis_index("c")
    sid = lax.axis_index("s")
    wid = sid * NC + cid
    col0 = wid * COLS_W

    pltpu.sync_copy(ea_hbm, tab_a)
    pltpu.sync_copy(eb_hbm, tab_b)
    s = jnp.zeros((L,), jnp.float32)
    for k in range(L):
        s = s + tab_a[k] + tab_b[k]
    s_ref[...] = s

    bufs = (buf0, buf1)
    sems = (sem0, sem1)

    def dma(c, buf, sem):
        return pltpu.make_async_copy(
            xt_hbm.at[pl.ds(c * RCHUNK, RCHUNK), pl.ds(col0, COLS_W)],
            buf, sem)

    dma(0, buf0, sem0).start()
    acc = jnp.zeros((L,), jnp.float32)
    for c in range(NCHUNK):
        buf, sem = bufs[c % 2], sems[c % 2]
        if c + 1 < NCHUNK:
            dma(c + 1, bufs[(c + 1) % 2], sems[(c + 1) % 2]).start()
        dma(c, buf, sem).wait()

        def body(r, a, buf=buf):
            for j in range(VROW):
                idx = buf[r, pl.ds(j * L, L)]
                a = a + plsc.load_gather(s_ref, [idx])
            return a

        acc = lax.fori_loop(0, RCHUNK, body, acc)

    acc_ref[0, :] = acc
    pltpu.sync_copy(acc_ref, out_hbm.at[pl.ds(wid, 1)])


def kernel(x, emb, emb2):
    xt = x.astype(jnp.int32).T
    ea = jnp.zeros((L, L), jnp.float32).at[:10, :10].set(emb.T)
    eb = jnp.zeros((L, L), jnp.float32).at[:10, :10].set(emb2.T)
    partials = _sc_sum(xt, ea, eb)
    return jnp.sum(partials)
